# Initial kernel scaffold; baseline (speedup 1.0000x reference)
import jax
import jax.numpy as jnp
from jax.experimental import pallas as pl


def kernel(input, positions, hidden, emb_table, pos_table):
    raise NotImplementedError("write your pallas kernel here")


# speedup vs baseline: 2.0574x; 2.0574x over previous
import functools

import jax
import jax.numpy as jnp
from jax import lax
from jax.experimental import pallas as pl
from jax.experimental.pallas import tpu as pltpu
from jax.experimental.pallas import tpu_sc as plsc

NC = 2
NS = 16
NW = NC * NS
L = 16

BATCH = 4096
SEQ = 200
HIDDEN = 64
NJ = HIDDEN // L

ROWS_PER_W = BATCH // NW
G = 4
CHUNK = G * SEQ
NCHUNKS = ROWS_PER_W // G
TOK_PER_W = ROWS_PER_W * SEQ

_mesh = plsc.VectorSubcoreMesh(
    core_axis_name="c", subcore_axis_name="s", num_cores=NC, num_subcores=NS
)


@functools.partial(
    pl.kernel,
    out_type=(
        jax.ShapeDtypeStruct((BATCH * SEQ, HIDDEN), jnp.float32),
        jax.ShapeDtypeStruct((BATCH, HIDDEN), jnp.float32),
    ),
    mesh=_mesh,
    compiler_params=pltpu.CompilerParams(use_tc_tiling_on_sc=False),
    scratch_types=[
        pltpu.VMEM((CHUNK,), jnp.int32),
        pltpu.VMEM((CHUNK,), jnp.int32),
        pltpu.VMEM((CHUNK, HIDDEN), jnp.float32),
        pltpu.VMEM((CHUNK, HIDDEN), jnp.float32),
        pltpu.VMEM((ROWS_PER_W, HIDDEN), jnp.float32),
    ],
)
def _encoder_sc(ids_hbm, pos_hbm, emb_hbm, pot_hbm, summed_hbm, pooled_hbm,
                ids_v, pos_v, e_buf, p_buf, pool_buf):
    wid = lax.axis_index("s") * NC + lax.axis_index("c")
    w_base = wid * TOK_PER_W
    inv_seq = jnp.float32(1.0 / SEQ)

    def chunk_body(c, carry):
        tok0 = pl.multiple_of(w_base + c * CHUNK, CHUNK)
        pltpu.sync_copy(ids_hbm.at[pl.ds(tok0, CHUNK)], ids_v)
        pltpu.sync_copy(pos_hbm.at[pl.ds(tok0, CHUNK)], pos_v)
        pltpu.sync_copy(emb_hbm.at[ids_v], e_buf)
        pltpu.sync_copy(pot_hbm.at[pos_v], p_buf)

        for g in range(G):
            def t_body(t, acc):
                row = g * SEQ + t
                new = []
                for j in range(NJ):
                    e = e_buf[row, pl.ds(j * L, L)]
                    p = p_buf[row, pl.ds(j * L, L)]
                    s = e + p
                    e_buf[row, pl.ds(j * L, L)] = s
                    new.append(acc[j] + s)
                return tuple(new)

            zeros = tuple(jnp.zeros((L,), jnp.float32) for _ in range(NJ))
            acc = lax.fori_loop(0, SEQ, t_body, zeros)
            prow = c * G + g
            for j in range(NJ):
                pool_buf[prow, pl.ds(j * L, L)] = acc[j] * inv_seq

        pltpu.sync_copy(e_buf, summed_hbm.at[pl.ds(tok0, CHUNK)])
        return carry

    lax.fori_loop(0, NCHUNKS, chunk_body, jnp.int32(0))
    pltpu.sync_copy(pool_buf, pooled_hbm.at[pl.ds(wid * ROWS_PER_W, ROWS_PER_W)])


def kernel(input, positions, hidden, emb_table, pos_table):
    del hidden
    ids = input.reshape(BATCH * SEQ)
    pos = positions.reshape(BATCH * SEQ)
    summed_flat, pooled = _encoder_sc(ids, pos, emb_table, pos_table)
    return (pooled[None], summed_flat.reshape(BATCH, SEQ, HIDDEN))
